# baseline (speedup 1.0000x reference)
import jax
import jax.numpy as jnp
from jax.experimental import pallas as pl
from jax.experimental.pallas import tpu as pltpu

K_NEIGHBORS = 10
N, KSHOT, Q = 100, 5, 15
D_IN, D_OUT = 2048, 1024
NQ = N * Q
NQP = 1536
NP = 128
ROWS = N * (KSHOT + Q)
BLK = 1536
NB = NQP // BLK
NEG = -1e30
BIG = 1e30


def _enc_kernel(x_ref, w_ref, b_ref, out_ref):
    out_ref[:] = (
        jnp.dot(x_ref[:], w_ref[:], preferred_element_type=jnp.float32) + b_ref[:]
    )


def _mega_kernel(sup_ref, q_ref, tao_ref, out_ref,
                 qn_scr, apn_scr, sim_scr, thrc_scr, thrr_scr):
    s = pl.program_id(0)

    @pl.when(s == 0)
    def _phase_proto():
        sup = sup_ref[:]
        qf = q_ref[:]

        proto = (
            sup[0 * NP:1 * NP] + sup[1 * NP:2 * NP] + sup[2 * NP:3 * NP]
            + sup[3 * NP:4 * NP] + sup[4 * NP:5 * NP]
        ) / float(KSHOT)
        pnorm = jnp.sqrt(jnp.sum(proto * proto, axis=1, keepdims=True))
        pn = proto / (pnorm + 1e-8)
        qnorm = jnp.sqrt(jnp.sum(qf * qf, axis=1, keepdims=True))
        qn = qf / (qnorm + 1e-8)
        qn_scr[:] = qn

        pre = jax.lax.dot_general(
            qn, pn, (((1,), (1,)), ((), ())), preferred_element_type=jnp.float32
        )
        colid = jax.lax.broadcasted_iota(jnp.int32, (NQP, NP), 1)
        rowid = jax.lax.broadcasted_iota(jnp.int32, (NQP, NP), 0)
        pre_m = jnp.where(colid < N, pre, NEG)
        rowmax = jnp.max(pre_m, axis=1, keepdims=True)
        idx = jnp.where(pre_m == rowmax, colid, jnp.int32(2**30))
        amin = jnp.min(idx, axis=1, keepdims=True)
        onehot = (colid == amin) & (rowid < NQ)
        exp_ref_w = jnp.where(onehot, jnp.exp(pre), 0.0)

        self_sim = jnp.sum(pn * pn, axis=1, keepdims=True)
        exp_self = jnp.exp(self_sim)
        ones = jnp.ones((NQP, 1), dtype=jnp.float32)
        denom = jax.lax.dot_general(
            exp_ref_w, ones, (((0,), (0,)), ((), ())),
            preferred_element_type=jnp.float32,
        ) + exp_self
        num = jax.lax.dot_general(
            exp_ref_w, qf, (((0,), (0,)), ((), ())),
            preferred_element_type=jnp.float32,
        ) + exp_self * proto
        ap = num / denom
        apnorm = jnp.sqrt(jnp.sum(ap * ap, axis=1, keepdims=True))
        apn_scr[:] = ap / (apnorm + 1e-8)

    @pl.when((s >= 1) & (s <= NB))
    def _phase_sim():
        t = s - 1
        a = qn_scr[pl.ds(t * BLK, BLK), :]
        b = qn_scr[:]
        sim = jax.lax.dot_general(
            a, b, (((1,), (1,)), ((), ())), preferred_element_type=jnp.float32
        )
        sim_scr[pl.ds(t * BLK, BLK), :] = sim
        colid = jax.lax.broadcasted_iota(jnp.int32, (BLK, NQP), 1)
        work = jnp.where(colid < NQ, sim, NEG)
        thr = jnp.full((BLK, 1), NEG, dtype=jnp.float32)
        for _ in range(K_NEIGHBORS):
            thr = jnp.max(work, axis=1, keepdims=True)
            work = jnp.where(work >= thr, NEG, work)
        rowg = t * BLK + jax.lax.broadcasted_iota(jnp.int32, (BLK, 1), 0)
        thr = jnp.where(rowg < NQ, thr, BIG)
        thrc_scr[pl.ds(t * BLK, BLK), :] = thr
        thrr_scr[:, pl.ds(t * BLK, BLK)] = jnp.transpose(thr, (1, 0))

    @pl.when(s > NB)
    def _phase_agg():
        j = s - NB - 1
        sim = sim_scr[pl.ds(j * BLK, BLK), :]
        ti = thrc_scr[pl.ds(j * BLK, BLK), :]
        tj = thrr_scr[:]
        w = jnp.where((sim >= ti) & (sim >= tj), jnp.exp(sim), 0.0)
        ssum = jnp.sum(w, axis=1, keepdims=True)
        ssum = jnp.where(ssum > 0.0, ssum, 1.0)
        aq = jnp.dot(w, q_ref[:], preferred_element_type=jnp.float32) / ssum
        anorm = jnp.sqrt(jnp.sum(aq * aq, axis=1, keepdims=True))
        aqn = aq / (anorm + 1e-8)
        out_ref[:] = tao_ref[0, 0] * jax.lax.dot_general(
            aqn, apn_scr[:], (((1,), (1,)), ((), ())),
            preferred_element_type=jnp.float32,
        )


def kernel(x, W_enc, b_enc, tao, n, k, q):
    f32 = jnp.float32

    feat = pl.pallas_call(
        _enc_kernel,
        grid=(8,),
        in_specs=[
            pl.BlockSpec((256, D_IN), lambda i: (i, 0)),
            pl.BlockSpec((D_IN, D_OUT), lambda i: (0, 0)),
            pl.BlockSpec((1, D_OUT), lambda i: (0, 0)),
        ],
        out_specs=pl.BlockSpec((256, D_OUT), lambda i: (i, 0)),
        out_shape=jax.ShapeDtypeStruct((ROWS, D_OUT), f32),
    )(x, W_enc, b_enc.reshape(1, D_OUT))

    f3 = feat.reshape(N, KSHOT + Q, D_OUT)
    sup3 = jnp.pad(f3[:, :KSHOT], ((0, NP - N), (0, 0), (0, 0)))
    sup = jnp.transpose(sup3, (1, 0, 2)).reshape(KSHOT * NP, D_OUT)
    qf = f3[:, KSHOT:].reshape(NQ, D_OUT)
    qf = jnp.pad(qf, ((0, NQP - NQ), (0, 0)))

    out = pl.pallas_call(
        _mega_kernel,
        grid=(2 * NB + 1,),
        in_specs=[
            pl.BlockSpec((KSHOT * NP, D_OUT), lambda i: (0, 0)),
            pl.BlockSpec((NQP, D_OUT), lambda i: (0, 0)),
            pl.BlockSpec((1, 1), lambda i: (0, 0)),
        ],
        out_specs=pl.BlockSpec(
            (BLK, NP), lambda i: (jnp.clip(i - NB - 1, 0, NB - 1), 0)
        ),
        out_shape=jax.ShapeDtypeStruct((NQP, NP), f32),
        scratch_shapes=[
            pltpu.VMEM((NQP, D_OUT), f32),
            pltpu.VMEM((NP, D_OUT), f32),
            pltpu.VMEM((NQP, NQP), f32),
            pltpu.VMEM((NQP, 1), f32),
            pltpu.VMEM((1, NQP), f32),
        ],
    )(sup, qf, tao.reshape(1, 1))

    return out[:NQ, :N]


# speedup vs baseline: 1.0231x; 1.0231x over previous
import jax
import jax.numpy as jnp
from jax.experimental import pallas as pl
from jax.experimental.pallas import tpu as pltpu

K_NEIGHBORS = 10
N, KSHOT, Q = 100, 5, 15
D_IN, D_OUT = 2048, 1024
NQ = N * Q
NQP = 1536
NP = 128
ROWS = N * (KSHOT + Q)
BLK = 768
NB = NQP // BLK
NEG = -1e30
BIG = 1e30


def _enc_kernel(x_ref, w_ref, b_ref, out_ref):
    out_ref[:] = (
        jnp.dot(x_ref[:], w_ref[:], preferred_element_type=jnp.float32) + b_ref[:]
    )


def _mega_kernel(sup_ref, q_ref, tao_ref, out_ref,
                 qn_scr, apn_scr, sim_scr, thrc_scr, thrr_scr):
    s = pl.program_id(0)

    @pl.when(s == 0)
    def _phase_proto():
        sup = sup_ref[:]
        qf = q_ref[:]

        proto = (
            sup[0 * NP:1 * NP] + sup[1 * NP:2 * NP] + sup[2 * NP:3 * NP]
            + sup[3 * NP:4 * NP] + sup[4 * NP:5 * NP]
        ) / float(KSHOT)
        pnorm = jnp.sqrt(jnp.sum(proto * proto, axis=1, keepdims=True))
        pn = proto / (pnorm + 1e-8)
        qnorm = jnp.sqrt(jnp.sum(qf * qf, axis=1, keepdims=True))
        qn = qf / (qnorm + 1e-8)
        qn_scr[:] = qn

        pre = jax.lax.dot_general(
            qn, pn, (((1,), (1,)), ((), ())), preferred_element_type=jnp.float32
        )
        colid = jax.lax.broadcasted_iota(jnp.int32, (NQP, NP), 1)
        rowid = jax.lax.broadcasted_iota(jnp.int32, (NQP, NP), 0)
        pre_m = jnp.where(colid < N, pre, NEG)
        rowmax = jnp.max(pre_m, axis=1, keepdims=True)
        idx = jnp.where(pre_m == rowmax, colid, jnp.int32(2**30))
        amin = jnp.min(idx, axis=1, keepdims=True)
        onehot = (colid == amin) & (rowid < NQ)
        exp_ref_w = jnp.where(onehot, jnp.exp(pre), 0.0)

        self_sim = jnp.sum(pn * pn, axis=1, keepdims=True)
        exp_self = jnp.exp(self_sim)
        ones = jnp.ones((NQP, 1), dtype=jnp.float32)
        denom = jax.lax.dot_general(
            exp_ref_w, ones, (((0,), (0,)), ((), ())),
            preferred_element_type=jnp.float32,
        ) + exp_self
        num = jax.lax.dot_general(
            exp_ref_w, qf, (((0,), (0,)), ((), ())),
            preferred_element_type=jnp.float32,
        ) + exp_self * proto
        ap = num / denom
        apnorm = jnp.sqrt(jnp.sum(ap * ap, axis=1, keepdims=True))
        apn_scr[:] = ap / (apnorm + 1e-8)

    @pl.when((s >= 1) & (s <= NB))
    def _phase_sim():
        t = s - 1
        a = qn_scr[pl.ds(t * BLK, BLK), :]
        b = qn_scr[:]
        sim = jax.lax.dot_general(
            a, b, (((1,), (1,)), ((), ())), preferred_element_type=jnp.float32
        )
        sim_scr[pl.ds(t * BLK, BLK), :] = sim
        colid = jax.lax.broadcasted_iota(jnp.int32, (BLK, NQP), 1)
        work = jnp.where(colid < NQ, sim, NEG)
        thr = jnp.full((BLK, 1), NEG, dtype=jnp.float32)
        for _ in range(K_NEIGHBORS):
            thr = jnp.max(work, axis=1, keepdims=True)
            work = jnp.where(work >= thr, NEG, work)
        rowg = t * BLK + jax.lax.broadcasted_iota(jnp.int32, (BLK, 1), 0)
        thr = jnp.where(rowg < NQ, thr, BIG)
        thrc_scr[pl.ds(t * BLK, BLK), :] = thr
        thrr_scr[:, pl.ds(t * BLK, BLK)] = jnp.transpose(thr, (1, 0))

    @pl.when(s > NB)
    def _phase_agg():
        j = s - NB - 1
        sim = sim_scr[pl.ds(j * BLK, BLK), :]
        ti = thrc_scr[pl.ds(j * BLK, BLK), :]
        tj = thrr_scr[:]
        w = jnp.where((sim >= ti) & (sim >= tj), jnp.exp(sim), 0.0)
        ssum = jnp.sum(w, axis=1, keepdims=True)
        ssum = jnp.where(ssum > 0.0, ssum, 1.0)
        aq = jnp.dot(w, q_ref[:], preferred_element_type=jnp.float32) / ssum
        anorm = jnp.sqrt(jnp.sum(aq * aq, axis=1, keepdims=True))
        aqn = aq / (anorm + 1e-8)
        out_ref[:] = tao_ref[0, 0] * jax.lax.dot_general(
            aqn, apn_scr[:], (((1,), (1,)), ((), ())),
            preferred_element_type=jnp.float32,
        )


def kernel(x, W_enc, b_enc, tao, n, k, q):
    f32 = jnp.float32

    feat = pl.pallas_call(
        _enc_kernel,
        grid=(4,),
        in_specs=[
            pl.BlockSpec((512, D_IN), lambda i: (i, 0)),
            pl.BlockSpec((D_IN, D_OUT), lambda i: (0, 0)),
            pl.BlockSpec((1, D_OUT), lambda i: (0, 0)),
        ],
        out_specs=pl.BlockSpec((512, D_OUT), lambda i: (i, 0)),
        out_shape=jax.ShapeDtypeStruct((ROWS, D_OUT), f32),
    )(x, W_enc, b_enc.reshape(1, D_OUT))

    f3 = feat.reshape(N, KSHOT + Q, D_OUT)
    sup3 = jnp.pad(f3[:, :KSHOT], ((0, NP - N), (0, 0), (0, 0)))
    sup = jnp.transpose(sup3, (1, 0, 2)).reshape(KSHOT * NP, D_OUT)
    qf = f3[:, KSHOT:].reshape(NQ, D_OUT)
    qf = jnp.pad(qf, ((0, NQP - NQ), (0, 0)))

    out = pl.pallas_call(
        _mega_kernel,
        grid=(2 * NB + 1,),
        in_specs=[
            pl.BlockSpec((KSHOT * NP, D_OUT), lambda i: (0, 0)),
            pl.BlockSpec((NQP, D_OUT), lambda i: (0, 0)),
            pl.BlockSpec((1, 1), lambda i: (0, 0)),
        ],
        out_specs=pl.BlockSpec(
            (BLK, NP), lambda i: (jnp.clip(i - NB - 1, 0, NB - 1), 0)
        ),
        out_shape=jax.ShapeDtypeStruct((NQP, NP), f32),
        scratch_shapes=[
            pltpu.VMEM((NQP, D_OUT), f32),
            pltpu.VMEM((NP, D_OUT), f32),
            pltpu.VMEM((NQP, NQP), f32),
            pltpu.VMEM((NQP, 1), f32),
            pltpu.VMEM((1, NQP), f32),
        ],
    )(sup, qf, tao.reshape(1, 1))

    return out[:NQ, :N]
